# baseline (speedup 1.0000x reference)
import functools

import jax
import jax.numpy as jnp
from jax import lax
from jax.experimental import pallas as pl
from jax.experimental.pallas import tpu as pltpu
from jax.experimental.pallas import tpu_sc as plsc

_N = 20000
_NPAD = 20480
_NCAND = 2048
_THR = 0.5
_BLK = 256
_NOUT = 100
_NTILE = 32
_CHUNK = _NPAD // _NTILE
_STEPS = _CHUNK // 16


def _thresh_core(bits_a_ref, bits_b_ref, meta_ref, counts_ref):
    bits = bits_a_ref[...]

    def srch(_, c):
        lo, hi = c
        mid = lo + lax.div(hi - lo, 2)
        cnt = jnp.sum(jnp.where(bits > mid, 1.0, 0.0))
        return jnp.where(cnt < float(_NCAND), lo, mid + 1), \
            jnp.where(cnt < float(_NCAND), mid, hi)

    lo, _ = lax.fori_loop(0, 31, srch, (jnp.int32(0), jnp.int32(0x7F000000)))
    g = jnp.sum(jnp.where(bits > lo, 1.0, 0.0))
    need_eq = float(_NCAND) - g

    lane = lax.broadcasted_iota(jnp.int32, (1, 128), 1)
    vstar = lax.bitcast_convert_type(jnp.full((1, 128), lo, jnp.int32),
                                     jnp.float32)
    meta = jnp.where(lane == 0, vstar,
                     jnp.where(lane == 1, g,
                               jnp.where(lane == 2, need_eq, 0.0)))
    meta_ref[...] = meta

    bb = bits_b_ref[...]
    gt_cnt = jnp.sum(jnp.where(bb > lo, 1.0, 0.0), axis=1, keepdims=True)
    eq_cnt = jnp.sum(jnp.where(bb == lo, 1.0, 0.0), axis=1, keepdims=True)
    cl = lax.broadcasted_iota(jnp.int32, (_NTILE, 8), 1)
    counts_ref[...] = jnp.where(cl == 0, gt_cnt,
                                jnp.where(cl == 1, eq_cnt, 0.0))


def _lane_gather(x, idx):
    return lax.gather(
        x, idx[:, None],
        lax.GatherDimensionNumbers(offset_dims=(), collapsed_slice_dims=(0,),
                                   start_index_map=(0,)),
        (1,), mode=lax.GatherScatterMode.PROMISE_IN_BOUNDS)


def _bcast16(x, lane):
    return _lane_gather(x, jnp.full((16,), lane, jnp.int32))


def _cumsum16(x, iot):
    y = x
    for s in (1, 2, 4, 8):
        sh = _lane_gather(y, jnp.maximum(iot - s, 0))
        y = y + jnp.where(iot >= s, sh, jnp.zeros_like(sh))
    return y


def _sc_compact(scores_hbm, boxflat_hbm, meta_hbm, counts_hbm, out_hbm,
                sc_v, bx_v, meta_v, counts_v, rec_v, dst_v, sem):
    wid = lax.axis_index("s") * 2 + lax.axis_index("c")
    base = wid * _CHUNK
    pltpu.sync_copy(scores_hbm.at[pl.ds(base, _CHUNK)], sc_v)
    for d in range(4):
        pltpu.sync_copy(boxflat_hbm.at[pl.ds(d * _NPAD + base, _CHUNK)],
                        bx_v.at[pl.ds(d * _CHUNK, _CHUNK)])
    pltpu.sync_copy(meta_hbm, meta_v)
    pltpu.sync_copy(counts_hbm, counts_v)

    iot = lax.broadcasted_iota(jnp.int32, (16,), 0)
    m0 = meta_v[pl.ds(0, 16)]
    vstar = _bcast16(m0, 0)
    g_tot = _bcast16(m0, 1).astype(jnp.int32)
    need_eq = _bcast16(m0, 2).astype(jnp.int32)

    zer = jnp.zeros((16,), jnp.int32)
    ga = plsc.load_gather(counts_v, [iot, zer])
    gb = plsc.load_gather(counts_v, [iot + 16, zer])
    ea = plsc.load_gather(counts_v, [iot, zer + 1])
    eb = plsc.load_gather(counts_v, [iot + 16, zer + 1])

    def _masked_total(v, limit):
        m = jnp.where(iot < limit, v, jnp.zeros_like(v))
        return _bcast16(_cumsum16(m, iot), 15)

    gt_run = (_masked_total(ga, wid) +
              _masked_total(gb, wid - 16)).astype(jnp.int32)
    eq_run = (_masked_total(ea, wid) +
              _masked_total(eb, wid - 16)).astype(jnp.int32)

    for k in range(_STEPS):
        off = k * 16
        s16 = sc_v[pl.ds(off, 16)]
        gt = s16 > vstar
        eq = s16 == vstar
        gt_i = gt.astype(jnp.int32)
        eq_i = eq.astype(jnp.int32)
        gt_cum = _cumsum16(gt_i, iot)
        eq_cum = _cumsum16(eq_i, iot)
        eq_rank = eq_run + eq_cum - 1
        eq_sel = eq & (eq_rank < need_eq)
        dst = jnp.where(gt, gt_run + gt_cum - 1,
                        jnp.where(eq_sel, g_tot + eq_rank,
                                  _NCAND + wid))
        gt_run = gt_run + _bcast16(gt_cum, 15)
        eq_run = eq_run + _bcast16(eq_cum, 15)

        row = off + iot
        idxf = (base + row).astype(jnp.float32)
        for c, val in ((0, bx_v[pl.ds(off, 16)]),
                       (1, bx_v[pl.ds(_CHUNK + off, 16)]),
                       (2, bx_v[pl.ds(2 * _CHUNK + off, 16)]),
                       (3, bx_v[pl.ds(3 * _CHUNK + off, 16)]),
                       (4, s16),
                       (5, idxf)):
            plsc.store_scatter(rec_v, [row, zer + c], val)
        dst_v[k // 8, pl.ds((k % 8) * 16, 16)] = dst

    cps = [pltpu.async_copy(rec_v.at[pl.ds(j * 128, 128)],
                            out_hbm.at[dst_v.at[j]], sem)
           for j in range(_CHUNK // 128)]
    for cp in cps:
        cp.wait()


def _nms_core(recs_ref, recsT_ref, out_ref, pa_scr, pb_scr, so_scr, soT_scr,
              rc_scr):
    nblk = _NCAND // _BLK

    def rnk(b, rrow):
        sl = pl.ds(b * _BLK, _BLK)
        s_col = recs_ref[sl, 4:5]
        i_col = recs_ref[sl, 5:6]
        s_row = recsT_ref[4:5, :]
        i_row = recsT_ref[5:6, :]
        c = jnp.where((s_col > s_row) |
                      ((s_col == s_row) & (i_col < i_row)), 1.0, 0.0)
        rc_scr[sl, :] = (float(_NCAND - 1) -
                         jnp.sum(c, axis=1, keepdims=True))
        return rrow + jnp.sum(c, axis=0, keepdims=True)

    rank_row = lax.fori_loop(0, nblk, rnk,
                             jnp.zeros((1, _NCAND), jnp.float32))
    rank_i = rank_row.astype(jnp.int32)

    def mkperm(b, carry):
        sl = pl.ds(b * _BLK, _BLK)
        isub = lax.broadcasted_iota(jnp.int32, (_BLK, _NCAND), 0) + b * _BLK
        ilane = lax.broadcasted_iota(jnp.int32, (_BLK, _NCAND), 1)
        pa_scr[sl, :] = jnp.where(
            isub == jnp.broadcast_to(rank_i, (_BLK, _NCAND)), 1.0, 0.0)
        rci = rc_scr[sl, :].astype(jnp.int32)
        pb_scr[sl, :] = jnp.where(ilane == rci, 1.0, 0.0)
        return carry

    lax.fori_loop(0, nblk, mkperm, 0)
    so_scr[...] = lax.dot_general(
        pa_scr[...], recs_ref[...], (((1,), (0,)), ((), ())),
        preferred_element_type=jnp.float32, precision=lax.Precision.HIGHEST)
    soT_scr[...] = lax.dot_general(
        recsT_ref[...], pb_scr[...], (((1,), (0,)), ((), ())),
        preferred_element_type=jnp.float32, precision=lax.Precision.HIGHEST)

    def blk(i, carry):
        bx = so_scr[pl.ds(i * _BLK, _BLK), :]
        x1c = bx[:, 0:1]
        y1c = bx[:, 1:2]
        x2c = bx[:, 2:3]
        y2c = bx[:, 3:4]
        area_c = (x2c - x1c) * (y2c - y1c)
        x1r = soT_scr[0:1, :]
        y1r = soT_scr[1:2, :]
        x2r = soT_scr[2:3, :]
        y2r = soT_scr[3:4, :]
        area_r = (x2r - x1r) * (y2r - y1r)
        w = jnp.maximum(jnp.minimum(x2c, x2r) - jnp.maximum(x1c, x1r), 0.0)
        h = jnp.maximum(jnp.minimum(y2c, y2r) - jnp.maximum(y1c, y1r), 0.0)
        inter = w * h
        union = area_c + area_r - inter
        iou = inter / jnp.maximum(union, 1e-9)
        jrow = lax.broadcasted_iota(jnp.int32, (_BLK, _NCAND), 0) + i * _BLK
        icol = lax.broadcasted_iota(jnp.int32, (_BLK, _NCAND), 1)
        pa_scr[pl.ds(i * _BLK, _BLK), :] = jnp.where(
            (iou > _THR) & (jrow < icol), 1.0, 0.0)
        pb_scr[pl.ds(i * _BLK, _BLK), :] = jnp.where(jrow <= icol, 1.0, 0.0)
        return carry

    lax.fori_loop(0, nblk, blk, 0)
    S = pa_scr[...]

    def cond(c):
        return c[1]

    def body(c):
        keep, _ = c
        cnt = lax.dot_general(keep, S, (((1,), (0,)), ((), ())),
                              preferred_element_type=jnp.float32)
        keep_new = jnp.where(cnt == 0.0, 1.0, 0.0)
        return keep_new, jnp.sum(jnp.abs(keep_new - keep)) > 0.0

    keep, _ = lax.while_loop(cond, body,
                             (jnp.ones((8, _NCAND), jnp.float32),
                              jnp.array(True)))

    Lm = pb_scr[...]
    kri = lax.dot_general(keep, Lm, (((1,), (0,)), ((), ())),
                          preferred_element_type=jnp.float32)
    sri = lax.dot_general(1.0 - keep, Lm, (((1,), (0,)), ((), ())),
                          preferred_element_type=jnp.float32)
    nk = jnp.sum(keep[0:1, :])
    rank_kept = kri - keep
    rank_supp = nk + (sri - (1.0 - keep))
    out_rank = jnp.where(keep > 0.5, rank_kept, rank_supp)
    oh_rank = out_rank[0:1, :].astype(jnp.int32)
    iota_r = lax.broadcasted_iota(jnp.int32, (128, _NCAND), 0)
    oh = jnp.where(
        iota_r == jnp.broadcast_to(oh_rank, (128, _NCAND)), 1.0, 0.0)
    out = lax.dot_general(oh, so_scr[...], (((1,), (0,)), ((), ())),
                          preferred_element_type=jnp.float32,
                          precision=lax.Precision.HIGHEST)
    nk_i = nk.astype(jnp.int32)
    rr = lax.broadcasted_iota(jnp.int32, (128, 16), 0)
    cc = lax.broadcasted_iota(jnp.int32, (128, 16), 1)
    out_ref[...] = jnp.where((rr >= nk_i) & (cc == 4), -jnp.inf, out)


def kernel(boxes, scores):
    s_pad = jnp.concatenate(
        [scores, jnp.full((_NPAD - _N,), -1.0, jnp.float32)])
    bits = lax.bitcast_convert_type(s_pad, jnp.int32)
    b_pad = jnp.concatenate(
        [boxes, jnp.zeros((_NPAD - _N, 4), jnp.float32)], axis=0)

    meta, counts = pl.pallas_call(
        _thresh_core,
        out_shape=(jax.ShapeDtypeStruct((1, 128), jnp.float32),
                   jax.ShapeDtypeStruct((_NTILE, 8), jnp.float32)),
    )(bits.reshape(8, _NPAD // 8), bits.reshape(_NTILE, _CHUNK))

    mesh = plsc.VectorSubcoreMesh(core_axis_name="c", subcore_axis_name="s")
    recs = functools.partial(
        pl.kernel, mesh=mesh,
        compiler_params=pltpu.CompilerParams(needs_layout_passes=False,
                                             use_tc_tiling_on_sc=False),
        out_type=jax.ShapeDtypeStruct((_NCAND + _NTILE, 16), jnp.float32),
        scratch_types=[
            pltpu.VMEM((_CHUNK,), jnp.float32),
            pltpu.VMEM((4 * _CHUNK,), jnp.float32),
            pltpu.VMEM((128,), jnp.float32),
            pltpu.VMEM((_NTILE, 8), jnp.float32),
            pltpu.VMEM((_CHUNK, 16), jnp.float32),
            pltpu.VMEM((_CHUNK // 128, 128), jnp.int32),
            pltpu.SemaphoreType.DMA,
        ],
    )(_sc_compact)(s_pad, b_pad.T.reshape(-1), meta.reshape(-1), counts)

    recs = recs[:_NCAND]
    out = pl.pallas_call(
        _nms_core,
        out_shape=jax.ShapeDtypeStruct((128, 16), jnp.float32),
        scratch_shapes=[
            pltpu.VMEM((_NCAND, _NCAND), jnp.float32),
            pltpu.VMEM((_NCAND, _NCAND), jnp.float32),
            pltpu.VMEM((_NCAND, 16), jnp.float32),
            pltpu.VMEM((16, _NCAND), jnp.float32),
            pltpu.VMEM((_NCAND, 1), jnp.float32),
        ],
    )(recs, recs.T)
    return out[:_NOUT, :4], out[:_NOUT, 4]


# speedup vs baseline: 1.1483x; 1.1483x over previous
import functools

import jax
import jax.numpy as jnp
from jax import lax
from jax.experimental import pallas as pl
from jax.experimental.pallas import tpu as pltpu
from jax.experimental.pallas import tpu_sc as plsc

_N = 20000
_NPAD = 20480
_NCAND = 2048
_THR = 0.5
_BLK = 256
_NOUT = 100
_NTILE = 32
_CHUNK = _NPAD // _NTILE
_STEPS = _CHUNK // 16


def _thresh_core(bits_a_ref, bits_b_ref, meta_ref, counts_ref):
    bits = bits_a_ref[...]

    def srch(_, c):
        lo, hi = c
        mid = lo + lax.div(hi - lo, 2)
        cnt = jnp.sum(jnp.where(bits > mid, 1.0, 0.0))
        return jnp.where(cnt < float(_NCAND), lo, mid + 1), \
            jnp.where(cnt < float(_NCAND), mid, hi)

    lo, _ = lax.fori_loop(0, 31, srch, (jnp.int32(0), jnp.int32(0x7F000000)))
    g = jnp.sum(jnp.where(bits > lo, 1.0, 0.0))
    need_eq = float(_NCAND) - g

    lane = lax.broadcasted_iota(jnp.int32, (1, 128), 1)
    vstar = lax.bitcast_convert_type(jnp.full((1, 128), lo, jnp.int32),
                                     jnp.float32)
    meta = jnp.where(lane == 0, vstar,
                     jnp.where(lane == 1, g,
                               jnp.where(lane == 2, need_eq, 0.0)))
    meta_ref[...] = meta

    bb = bits_b_ref[...]
    gt_cnt = jnp.sum(jnp.where(bb > lo, 1.0, 0.0), axis=1, keepdims=True)
    eq_cnt = jnp.sum(jnp.where(bb == lo, 1.0, 0.0), axis=1, keepdims=True)
    cl = lax.broadcasted_iota(jnp.int32, (_NTILE, 8), 1)
    counts_ref[...] = jnp.where(cl == 0, gt_cnt,
                                jnp.where(cl == 1, eq_cnt, 0.0))


def _lane_gather(x, idx):
    return lax.gather(
        x, idx[:, None],
        lax.GatherDimensionNumbers(offset_dims=(), collapsed_slice_dims=(0,),
                                   start_index_map=(0,)),
        (1,), mode=lax.GatherScatterMode.PROMISE_IN_BOUNDS)


def _bcast16(x, lane):
    return _lane_gather(x, jnp.full((16,), lane, jnp.int32))


def _sc_compact(scores_hbm, boxflat_hbm, meta_hbm, counts_hbm, out_hbm,
                sc_v, bx_v, meta_v, counts_v, rec_v, dst_v, sem):
    wid = lax.axis_index("s") * 2 + lax.axis_index("c")
    base = wid * _CHUNK
    pltpu.sync_copy(scores_hbm.at[pl.ds(base, _CHUNK)], sc_v)
    for d in range(4):
        pltpu.sync_copy(boxflat_hbm.at[pl.ds(d * _NPAD + base, _CHUNK)],
                        bx_v.at[pl.ds(d * _CHUNK, _CHUNK)])
    pltpu.sync_copy(meta_hbm, meta_v)
    pltpu.sync_copy(counts_hbm, counts_v)

    iot = lax.broadcasted_iota(jnp.int32, (16,), 0)
    m0 = meta_v[pl.ds(0, 16)]
    vstar = _bcast16(m0, 0)
    g_tot = _bcast16(m0, 1).astype(jnp.int32)
    need_eq = _bcast16(m0, 2).astype(jnp.int32)

    zer = jnp.zeros((16,), jnp.int32)
    ga = plsc.load_gather(counts_v, [iot, zer])
    gb = plsc.load_gather(counts_v, [iot + 16, zer])
    ea = plsc.load_gather(counts_v, [iot, zer + 1])
    eb = plsc.load_gather(counts_v, [iot + 16, zer + 1])

    def _masked_total(v, limit):
        m = jnp.where(iot < limit, v, jnp.zeros_like(v))
        return _bcast16(plsc.cumsum(m), 15)

    gt_run = (_masked_total(ga, wid) +
              _masked_total(gb, wid - 16)).astype(jnp.int32)
    eq_run = (_masked_total(ea, wid) +
              _masked_total(eb, wid - 16)).astype(jnp.int32)

    for k in range(_STEPS):
        off = k * 16
        s16 = sc_v[pl.ds(off, 16)]
        gt = s16 > vstar
        eq = s16 == vstar
        gt_i = gt.astype(jnp.int32)
        eq_i = eq.astype(jnp.int32)
        gt_cum = plsc.cumsum(gt_i)
        eq_cum = plsc.cumsum(eq_i)
        eq_rank = eq_run + eq_cum - 1
        eq_sel = eq & (eq_rank < need_eq)
        dst = jnp.where(gt, gt_run + gt_cum - 1,
                        jnp.where(eq_sel, g_tot + eq_rank,
                                  _NCAND + wid))
        gt_run = gt_run + _bcast16(gt_cum, 15)
        eq_run = eq_run + _bcast16(eq_cum, 15)

        row = off + iot
        idxf = (base + row).astype(jnp.float32)
        for c, val in ((0, bx_v[pl.ds(off, 16)]),
                       (1, bx_v[pl.ds(_CHUNK + off, 16)]),
                       (2, bx_v[pl.ds(2 * _CHUNK + off, 16)]),
                       (3, bx_v[pl.ds(3 * _CHUNK + off, 16)]),
                       (4, s16),
                       (5, idxf)):
            plsc.store_scatter(rec_v, [row, zer + c], val)
        dst_v[k // 8, pl.ds((k % 8) * 16, 16)] = dst

    cps = [pltpu.async_copy(rec_v.at[pl.ds(j * 128, 128)],
                            out_hbm.at[dst_v.at[j]], sem)
           for j in range(_CHUNK // 128)]
    for cp in cps:
        cp.wait()


def _nms_core(recs_ref, recsT_ref, out_ref, pa_scr, pb_scr, so_scr, soT_scr):
    nblk = _NCAND // _BLK

    def rnk(b, rrow):
        sl = pl.ds(b * _BLK, _BLK)
        s_col = recs_ref[sl, 4:5]
        i_col = recs_ref[sl, 5:6]
        s_row = recsT_ref[4:5, :]
        i_row = recsT_ref[5:6, :]
        c = jnp.where((s_col > s_row) |
                      ((s_col == s_row) & (i_col < i_row)), 1.0, 0.0)
        return rrow + jnp.sum(c, axis=0, keepdims=True)

    rank_row = lax.fori_loop(0, nblk, rnk,
                             jnp.zeros((1, _NCAND), jnp.float32))
    rank_i = rank_row.astype(jnp.int32)

    def mkperm(b, carry):
        sl = pl.ds(b * _BLK, _BLK)
        isub = lax.broadcasted_iota(jnp.int32, (_BLK, _NCAND), 0) + b * _BLK
        pa_scr[sl, :] = jnp.where(
            isub == jnp.broadcast_to(rank_i, (_BLK, _NCAND)),
            1.0, 0.0).astype(jnp.bfloat16)
        return carry

    lax.fori_loop(0, nblk, mkperm, 0)
    recs = recs_ref[...]
    r_hi = recs.astype(jnp.bfloat16)
    rem = recs - r_hi.astype(jnp.float32)
    r_mid = rem.astype(jnp.bfloat16)
    r_lo = (rem - r_mid.astype(jnp.float32)).astype(jnp.bfloat16)
    pa = pa_scr[...]
    dn = (((1,), (0,)), ((), ()))
    sorted_v = (
        lax.dot_general(pa, r_hi, dn, preferred_element_type=jnp.float32) +
        lax.dot_general(pa, r_mid, dn, preferred_element_type=jnp.float32) +
        lax.dot_general(pa, r_lo, dn, preferred_element_type=jnp.float32))
    so_scr[...] = sorted_v
    soT_scr[0:4, :] = jnp.transpose(sorted_v[:, 0:4])

    def blk(i, carry):
        bx = so_scr[pl.ds(i * _BLK, _BLK), :]
        x1c = bx[:, 0:1]
        y1c = bx[:, 1:2]
        x2c = bx[:, 2:3]
        y2c = bx[:, 3:4]
        area_c = (x2c - x1c) * (y2c - y1c)
        x1r = soT_scr[0:1, :]
        y1r = soT_scr[1:2, :]
        x2r = soT_scr[2:3, :]
        y2r = soT_scr[3:4, :]
        area_r = (x2r - x1r) * (y2r - y1r)
        w = jnp.maximum(jnp.minimum(x2c, x2r) - jnp.maximum(x1c, x1r), 0.0)
        h = jnp.maximum(jnp.minimum(y2c, y2r) - jnp.maximum(y1c, y1r), 0.0)
        inter = w * h
        union = area_c + area_r - inter
        iou = inter / jnp.maximum(union, 1e-9)
        jrow = lax.broadcasted_iota(jnp.int32, (_BLK, _NCAND), 0) + i * _BLK
        icol = lax.broadcasted_iota(jnp.int32, (_BLK, _NCAND), 1)
        pa_scr[pl.ds(i * _BLK, _BLK), :] = jnp.where(
            (iou > _THR) & (jrow < icol), 1.0, 0.0).astype(jnp.bfloat16)
        pb_scr[pl.ds(i * _BLK, _BLK), :] = jnp.where(
            jrow <= icol, 1.0, 0.0).astype(jnp.bfloat16)
        return carry

    lax.fori_loop(0, nblk, blk, 0)
    S = pa_scr[...]

    def cond(c):
        return c[1]

    def body(c):
        keep, _ = c
        cnt = lax.dot_general(keep.astype(jnp.bfloat16), S,
                              (((1,), (0,)), ((), ())),
                              preferred_element_type=jnp.float32)
        keep_new = jnp.where(cnt == 0.0, 1.0, 0.0)
        return keep_new, jnp.sum(jnp.abs(keep_new - keep)) > 0.0

    keep, _ = lax.while_loop(cond, body,
                             (jnp.ones((8, _NCAND), jnp.float32),
                              jnp.array(True)))

    Lm = pb_scr[...]
    kri = lax.dot_general(keep.astype(jnp.bfloat16), Lm,
                          (((1,), (0,)), ((), ())),
                          preferred_element_type=jnp.float32)
    sri = lax.dot_general((1.0 - keep).astype(jnp.bfloat16), Lm,
                          (((1,), (0,)), ((), ())),
                          preferred_element_type=jnp.float32)
    nk = jnp.sum(keep[0:1, :])
    rank_kept = kri - keep
    rank_supp = nk + (sri - (1.0 - keep))
    out_rank = jnp.where(keep > 0.5, rank_kept, rank_supp)
    oh_rank = out_rank[0:1, :].astype(jnp.int32)
    iota_r = lax.broadcasted_iota(jnp.int32, (128, _NCAND), 0)
    oh = jnp.where(
        iota_r == jnp.broadcast_to(oh_rank, (128, _NCAND)), 1.0, 0.0)
    out = lax.dot_general(oh, so_scr[...], (((1,), (0,)), ((), ())),
                          preferred_element_type=jnp.float32,
                          precision=lax.Precision.HIGHEST)
    nk_i = nk.astype(jnp.int32)
    rr = lax.broadcasted_iota(jnp.int32, (128, 16), 0)
    cc = lax.broadcasted_iota(jnp.int32, (128, 16), 1)
    out_ref[...] = jnp.where((rr >= nk_i) & (cc == 4), -jnp.inf, out)


def kernel(boxes, scores):
    s_pad = jnp.concatenate(
        [scores, jnp.full((_NPAD - _N,), -1.0, jnp.float32)])
    bits = lax.bitcast_convert_type(s_pad, jnp.int32)
    b_pad = jnp.concatenate(
        [boxes, jnp.zeros((_NPAD - _N, 4), jnp.float32)], axis=0)

    meta, counts = pl.pallas_call(
        _thresh_core,
        out_shape=(jax.ShapeDtypeStruct((1, 128), jnp.float32),
                   jax.ShapeDtypeStruct((_NTILE, 8), jnp.float32)),
    )(bits.reshape(8, _NPAD // 8), bits.reshape(_NTILE, _CHUNK))

    mesh = plsc.VectorSubcoreMesh(core_axis_name="c", subcore_axis_name="s")
    recs = functools.partial(
        pl.kernel, mesh=mesh,
        compiler_params=pltpu.CompilerParams(needs_layout_passes=False,
                                             use_tc_tiling_on_sc=False),
        out_type=jax.ShapeDtypeStruct((_NCAND + _NTILE, 16), jnp.float32),
        scratch_types=[
            pltpu.VMEM((_CHUNK,), jnp.float32),
            pltpu.VMEM((4 * _CHUNK,), jnp.float32),
            pltpu.VMEM((128,), jnp.float32),
            pltpu.VMEM((_NTILE, 8), jnp.float32),
            pltpu.VMEM((_CHUNK, 16), jnp.float32),
            pltpu.VMEM((_CHUNK // 128, 128), jnp.int32),
            pltpu.SemaphoreType.DMA,
        ],
    )(_sc_compact)(s_pad, b_pad.T.reshape(-1), meta.reshape(-1), counts)

    recs = recs[:_NCAND]
    out = pl.pallas_call(
        _nms_core,
        out_shape=jax.ShapeDtypeStruct((128, 16), jnp.float32),
        scratch_shapes=[
            pltpu.VMEM((_NCAND, _NCAND), jnp.bfloat16),
            pltpu.VMEM((_NCAND, _NCAND), jnp.bfloat16),
            pltpu.VMEM((_NCAND, 16), jnp.float32),
            pltpu.VMEM((8, _NCAND), jnp.float32),
        ],
    )(recs, recs.T)
    return out[:_NOUT, :4], out[:_NOUT, 4]


# speedup vs baseline: 1.2053x; 1.0497x over previous
import functools

import jax
import jax.numpy as jnp
from jax import lax
from jax.experimental import pallas as pl
from jax.experimental.pallas import tpu as pltpu
from jax.experimental.pallas import tpu_sc as plsc

_N = 20000
_NPAD = 20480
_NCAND = 2048
_THR = 0.5
_BLK = 256
_NOUT = 100
_NTILE = 32
_CHUNK = _NPAD // _NTILE
_STEPS = _CHUNK // 16


def _thresh_core(bits_a_ref, bits_b_ref, meta_ref, counts_ref):
    bits = bits_a_ref[...]

    def srch(_, c):
        lo, hi = c
        mid = lo + lax.div(hi - lo, 2)
        cnt = jnp.sum(jnp.where(bits > mid, 1.0, 0.0))
        return jnp.where(cnt < float(_NCAND), lo, mid + 1), \
            jnp.where(cnt < float(_NCAND), mid, hi)

    lo, _ = lax.fori_loop(0, 31, srch, (jnp.int32(0), jnp.int32(0x7F000000)))
    g = jnp.sum(jnp.where(bits > lo, 1.0, 0.0))
    need_eq = float(_NCAND) - g

    lane = lax.broadcasted_iota(jnp.int32, (1, 128), 1)
    vstar = lax.bitcast_convert_type(jnp.full((1, 128), lo, jnp.int32),
                                     jnp.float32)
    meta = jnp.where(lane == 0, vstar,
                     jnp.where(lane == 1, g,
                               jnp.where(lane == 2, need_eq, 0.0)))
    meta_ref[...] = meta

    bb = bits_b_ref[...]
    gt_cnt = jnp.sum(jnp.where(bb > lo, 1.0, 0.0), axis=1, keepdims=True)
    eq_cnt = jnp.sum(jnp.where(bb == lo, 1.0, 0.0), axis=1, keepdims=True)
    cl = lax.broadcasted_iota(jnp.int32, (_NTILE, 8), 1)
    counts_ref[...] = jnp.where(cl == 0, gt_cnt,
                                jnp.where(cl == 1, eq_cnt, 0.0))


def _lane_gather(x, idx):
    return lax.gather(
        x, idx[:, None],
        lax.GatherDimensionNumbers(offset_dims=(), collapsed_slice_dims=(0,),
                                   start_index_map=(0,)),
        (1,), mode=lax.GatherScatterMode.PROMISE_IN_BOUNDS)


def _bcast16(x, lane):
    return _lane_gather(x, jnp.full((16,), lane, jnp.int32))


def _sc_compact(scores_hbm, boxflat_hbm, meta_hbm, counts_hbm, out_hbm,
                sc_v, bx_v, meta_v, counts_v, rec_v, dst_v, sem):
    wid = lax.axis_index("s") * 2 + lax.axis_index("c")
    base = wid * _CHUNK
    pltpu.sync_copy(scores_hbm.at[pl.ds(base, _CHUNK)], sc_v)
    for d in range(4):
        pltpu.sync_copy(boxflat_hbm.at[pl.ds(d * _NPAD + base, _CHUNK)],
                        bx_v.at[pl.ds(d * _CHUNK, _CHUNK)])
    pltpu.sync_copy(meta_hbm, meta_v)
    pltpu.sync_copy(counts_hbm, counts_v)

    iot = lax.broadcasted_iota(jnp.int32, (16,), 0)
    m0 = meta_v[pl.ds(0, 16)]
    vstar = _bcast16(m0, 0)
    g_tot = _bcast16(m0, 1).astype(jnp.int32)
    need_eq = _bcast16(m0, 2).astype(jnp.int32)

    zer = jnp.zeros((16,), jnp.int32)
    ga = plsc.load_gather(counts_v, [iot, zer])
    gb = plsc.load_gather(counts_v, [iot + 16, zer])
    ea = plsc.load_gather(counts_v, [iot, zer + 1])
    eb = plsc.load_gather(counts_v, [iot + 16, zer + 1])

    def _masked_total(v, limit):
        m = jnp.where(iot < limit, v, jnp.zeros_like(v))
        return _bcast16(plsc.cumsum(m), 15)

    gt_run = (_masked_total(ga, wid) +
              _masked_total(gb, wid - 16)).astype(jnp.int32)
    eq_run = (_masked_total(ea, wid) +
              _masked_total(eb, wid - 16)).astype(jnp.int32)

    for k in range(_STEPS):
        off = k * 16
        s16 = sc_v[pl.ds(off, 16)]
        gt = s16 > vstar
        eq = s16 == vstar
        gt_i = gt.astype(jnp.int32)
        eq_i = eq.astype(jnp.int32)
        gt_cum = plsc.cumsum(gt_i)
        eq_cum = plsc.cumsum(eq_i)
        eq_rank = eq_run + eq_cum - 1
        eq_sel = eq & (eq_rank < need_eq)
        dst = jnp.where(gt, gt_run + gt_cum - 1,
                        jnp.where(eq_sel, g_tot + eq_rank,
                                  _NCAND + wid))
        gt_run = gt_run + _bcast16(gt_cum, 15)
        eq_run = eq_run + _bcast16(eq_cum, 15)

        row = off + iot
        idxf = (base + row).astype(jnp.float32)
        for c, val in ((0, bx_v[pl.ds(off, 16)]),
                       (1, bx_v[pl.ds(_CHUNK + off, 16)]),
                       (2, bx_v[pl.ds(2 * _CHUNK + off, 16)]),
                       (3, bx_v[pl.ds(3 * _CHUNK + off, 16)]),
                       (4, s16),
                       (5, idxf)):
            plsc.store_scatter(rec_v, [row, zer + c], val)
        dst_v[k // 8, pl.ds((k % 8) * 16, 16)] = dst

    cps = [pltpu.async_copy(rec_v.at[pl.ds(j * 128, 128)],
                            out_hbm.at[dst_v.at[j]], sem)
           for j in range(_CHUNK // 128)]
    for cp in cps:
        cp.wait()


def _nms_core(recs_ref, out_ref, pa_scr, pb_scr, so_scr, soT_scr):
    nblk = _NCAND // _BLK
    si_rows = jnp.transpose(recs_ref[...][:, 4:6])

    def rnk(b, rrow):
        sl = pl.ds(b * _BLK, _BLK)
        s_col = recs_ref[sl, 4:5]
        i_col = recs_ref[sl, 5:6]
        s_row = si_rows[0:1, :]
        i_row = si_rows[1:2, :]
        c = jnp.where((s_col > s_row) |
                      ((s_col == s_row) & (i_col < i_row)), 1.0, 0.0)
        return rrow + jnp.sum(c, axis=0, keepdims=True)

    rank_row = lax.fori_loop(0, nblk, rnk,
                             jnp.zeros((1, _NCAND), jnp.float32))
    rank_i = rank_row.astype(jnp.int32)

    def mkperm(b, carry):
        sl = pl.ds(b * _BLK, _BLK)
        isub = lax.broadcasted_iota(jnp.int32, (_BLK, _NCAND), 0) + b * _BLK
        pa_scr[sl, :] = jnp.where(
            isub == jnp.broadcast_to(rank_i, (_BLK, _NCAND)),
            1.0, 0.0).astype(jnp.bfloat16)
        return carry

    lax.fori_loop(0, nblk, mkperm, 0)
    recs = recs_ref[...]
    r_hi = recs.astype(jnp.bfloat16)
    rem = recs - r_hi.astype(jnp.float32)
    r_mid = rem.astype(jnp.bfloat16)
    r_lo = (rem - r_mid.astype(jnp.float32)).astype(jnp.bfloat16)
    pa = pa_scr[...]
    dn = (((1,), (0,)), ((), ()))
    sorted_v = (
        lax.dot_general(pa, r_hi, dn, preferred_element_type=jnp.float32) +
        lax.dot_general(pa, r_mid, dn, preferred_element_type=jnp.float32) +
        lax.dot_general(pa, r_lo, dn, preferred_element_type=jnp.float32))
    so_scr[...] = sorted_v
    soT_scr[0:4, :] = jnp.transpose(sorted_v[:, 0:4])

    for rb in range(nblk):
        c0 = rb * _BLK
        cw = _NCAND - c0
        rsl = pl.ds(c0, _BLK)
        csl = pl.ds(c0, cw)
        bx = so_scr[rsl, :]
        x1c = bx[:, 0:1]
        y1c = bx[:, 1:2]
        x2c = bx[:, 2:3]
        y2c = bx[:, 3:4]
        area_c = (x2c - x1c) * (y2c - y1c)
        x1r = soT_scr[0:1, csl]
        y1r = soT_scr[1:2, csl]
        x2r = soT_scr[2:3, csl]
        y2r = soT_scr[3:4, csl]
        area_r = (x2r - x1r) * (y2r - y1r)
        w = jnp.maximum(jnp.minimum(x2c, x2r) - jnp.maximum(x1c, x1r), 0.0)
        h = jnp.maximum(jnp.minimum(y2c, y2r) - jnp.maximum(y1c, y1r), 0.0)
        inter = w * h
        union = area_c + area_r - inter
        iou = inter / jnp.maximum(union, 1e-9)
        jrow = lax.broadcasted_iota(jnp.int32, (_BLK, cw), 0) + c0
        icol = lax.broadcasted_iota(jnp.int32, (_BLK, cw), 1) + c0
        pa_scr[rsl, csl] = jnp.where(
            (iou > _THR) & (jrow < icol), 1.0, 0.0).astype(jnp.bfloat16)
        pb_scr[rsl, csl] = jnp.where(
            jrow <= icol, 1.0, 0.0).astype(jnp.bfloat16)
        if c0:
            zer = jnp.zeros((_BLK, c0), jnp.bfloat16)
            pa_scr[rsl, pl.ds(0, c0)] = zer
            pb_scr[rsl, pl.ds(0, c0)] = zer

    S = pa_scr[...]

    def cond(c):
        return c[1]

    def body(c):
        keep, _ = c
        cnt = lax.dot_general(keep.astype(jnp.bfloat16), S,
                              (((1,), (0,)), ((), ())),
                              preferred_element_type=jnp.float32)
        keep_new = jnp.where(cnt == 0.0, 1.0, 0.0)
        return keep_new, jnp.sum(jnp.abs(keep_new - keep)) > 0.0

    keep, _ = lax.while_loop(cond, body,
                             (jnp.ones((8, _NCAND), jnp.float32),
                              jnp.array(True)))

    Lm = pb_scr[...]
    kri = lax.dot_general(keep.astype(jnp.bfloat16), Lm,
                          (((1,), (0,)), ((), ())),
                          preferred_element_type=jnp.float32)
    sri = lax.dot_general((1.0 - keep).astype(jnp.bfloat16), Lm,
                          (((1,), (0,)), ((), ())),
                          preferred_element_type=jnp.float32)
    nk = jnp.sum(keep[0:1, :])
    rank_kept = kri - keep
    rank_supp = nk + (sri - (1.0 - keep))
    out_rank = jnp.where(keep > 0.5, rank_kept, rank_supp)
    oh_rank = out_rank[0:1, :].astype(jnp.int32)
    iota_r = lax.broadcasted_iota(jnp.int32, (128, _NCAND), 0)
    oh = jnp.where(
        iota_r == jnp.broadcast_to(oh_rank, (128, _NCAND)), 1.0, 0.0)
    out = lax.dot_general(oh, so_scr[...], (((1,), (0,)), ((), ())),
                          preferred_element_type=jnp.float32,
                          precision=lax.Precision.HIGHEST)
    nk_i = nk.astype(jnp.int32)
    rr = lax.broadcasted_iota(jnp.int32, (128, 16), 0)
    cc = lax.broadcasted_iota(jnp.int32, (128, 16), 1)
    out_ref[...] = jnp.where((rr >= nk_i) & (cc == 4), -jnp.inf, out)


def kernel(boxes, scores):
    s_pad = jnp.concatenate(
        [scores, jnp.full((_NPAD - _N,), -1.0, jnp.float32)])
    bits = lax.bitcast_convert_type(s_pad, jnp.int32)
    b_pad = jnp.concatenate(
        [boxes, jnp.zeros((_NPAD - _N, 4), jnp.float32)], axis=0)

    meta, counts = pl.pallas_call(
        _thresh_core,
        out_shape=(jax.ShapeDtypeStruct((1, 128), jnp.float32),
                   jax.ShapeDtypeStruct((_NTILE, 8), jnp.float32)),
    )(bits.reshape(8, _NPAD // 8), bits.reshape(_NTILE, _CHUNK))

    mesh = plsc.VectorSubcoreMesh(core_axis_name="c", subcore_axis_name="s")
    recs = functools.partial(
        pl.kernel, mesh=mesh,
        compiler_params=pltpu.CompilerParams(needs_layout_passes=False,
                                             use_tc_tiling_on_sc=False),
        out_type=jax.ShapeDtypeStruct((_NCAND + _NTILE, 16), jnp.float32),
        scratch_types=[
            pltpu.VMEM((_CHUNK,), jnp.float32),
            pltpu.VMEM((4 * _CHUNK,), jnp.float32),
            pltpu.VMEM((128,), jnp.float32),
            pltpu.VMEM((_NTILE, 8), jnp.float32),
            pltpu.VMEM((_CHUNK, 16), jnp.float32),
            pltpu.VMEM((_CHUNK // 128, 128), jnp.int32),
            pltpu.SemaphoreType.DMA,
        ],
    )(_sc_compact)(s_pad, b_pad.T.reshape(-1), meta.reshape(-1), counts)

    recs = recs[:_NCAND]
    out = pl.pallas_call(
        _nms_core,
        out_shape=jax.ShapeDtypeStruct((128, 16), jnp.float32),
        scratch_shapes=[
            pltpu.VMEM((_NCAND, _NCAND), jnp.bfloat16),
            pltpu.VMEM((_NCAND, _NCAND), jnp.bfloat16),
            pltpu.VMEM((_NCAND, 16), jnp.float32),
            pltpu.VMEM((8, _NCAND), jnp.float32),
        ],
    )(recs)
    return out[:_NOUT, :4], out[:_NOUT, 4]


# speedup vs baseline: 1.2334x; 1.0233x over previous
import functools

import jax
import jax.numpy as jnp
from jax import lax
from jax.experimental import pallas as pl
from jax.experimental.pallas import tpu as pltpu
from jax.experimental.pallas import tpu_sc as plsc

_N = 20000
_NPAD = 20480
_NCAND = 2048
_THR = 0.5
_BLK = 256
_NOUT = 100
_NTILE = 32
_CHUNK = _NPAD // _NTILE
_STEPS = _CHUNK // 16


def _thresh_core(bits_a_ref, bits_b_ref, meta_ref, counts_ref):
    bits = bits_a_ref[...]

    def srch(_, c):
        lo, hi = c
        mid = lo + lax.div(hi - lo, 2)
        cnt = jnp.sum(jnp.where(bits > mid, 1.0, 0.0))
        return jnp.where(cnt < float(_NCAND), lo, mid + 1), \
            jnp.where(cnt < float(_NCAND), mid, hi)

    lo, _ = lax.fori_loop(0, 31, srch, (jnp.int32(0), jnp.int32(0x7F000000)))
    g = jnp.sum(jnp.where(bits > lo, 1.0, 0.0))
    need_eq = float(_NCAND) - g

    lane = lax.broadcasted_iota(jnp.int32, (1, 128), 1)
    vstar = lax.bitcast_convert_type(jnp.full((1, 128), lo, jnp.int32),
                                     jnp.float32)
    meta = jnp.where(lane == 0, vstar,
                     jnp.where(lane == 1, g,
                               jnp.where(lane == 2, need_eq, 0.0)))
    meta_ref[...] = meta

    bb = bits_b_ref[...]
    gt_cnt = jnp.sum(jnp.where(bb > lo, 1.0, 0.0), axis=1, keepdims=True)
    eq_cnt = jnp.sum(jnp.where(bb == lo, 1.0, 0.0), axis=1, keepdims=True)
    cl = lax.broadcasted_iota(jnp.int32, (_NTILE, 8), 1)
    counts_ref[...] = jnp.where(cl == 0, gt_cnt,
                                jnp.where(cl == 1, eq_cnt, 0.0))


def _lane_gather(x, idx):
    return lax.gather(
        x, idx[:, None],
        lax.GatherDimensionNumbers(offset_dims=(), collapsed_slice_dims=(0,),
                                   start_index_map=(0,)),
        (1,), mode=lax.GatherScatterMode.PROMISE_IN_BOUNDS)


def _bcast16(x, lane):
    return _lane_gather(x, jnp.full((16,), lane, jnp.int32))


def _sc_compact(scores_hbm, boxflat_hbm, meta_hbm, counts_hbm, out_hbm,
                sc_v, bx_v, meta_v, counts_v, rec_v, dst_v, sem):
    wid = lax.axis_index("s") * 2 + lax.axis_index("c")
    base = wid * _CHUNK
    lds = [pltpu.async_copy(scores_hbm.at[pl.ds(base, _CHUNK)], sc_v, sem),
           pltpu.async_copy(meta_hbm, meta_v, sem),
           pltpu.async_copy(counts_hbm, counts_v, sem)]
    lds += [pltpu.async_copy(boxflat_hbm.at[pl.ds(d * _NPAD + base, _CHUNK)],
                             bx_v.at[pl.ds(d * _CHUNK, _CHUNK)], sem)
            for d in range(4)]
    for cp in lds:
        cp.wait()

    iot = lax.broadcasted_iota(jnp.int32, (16,), 0)
    m0 = meta_v[pl.ds(0, 16)]
    vstar = _bcast16(m0, 0)
    g_tot = _bcast16(m0, 1).astype(jnp.int32)
    need_eq = _bcast16(m0, 2).astype(jnp.int32)

    zer = jnp.zeros((16,), jnp.int32)
    ga = plsc.load_gather(counts_v, [iot, zer])
    gb = plsc.load_gather(counts_v, [iot + 16, zer])
    ea = plsc.load_gather(counts_v, [iot, zer + 1])
    eb = plsc.load_gather(counts_v, [iot + 16, zer + 1])

    def _masked_total(v, limit):
        m = jnp.where(iot < limit, v, jnp.zeros_like(v))
        return _bcast16(plsc.cumsum(m), 15)

    gt_run = (_masked_total(ga, wid) +
              _masked_total(gb, wid - 16)).astype(jnp.int32)
    eq_run = (_masked_total(ea, wid) +
              _masked_total(eb, wid - 16)).astype(jnp.int32)

    for k in range(_STEPS):
        off = k * 16
        s16 = sc_v[pl.ds(off, 16)]
        gt = s16 > vstar
        eq = s16 == vstar
        gt_i = gt.astype(jnp.int32)
        eq_i = eq.astype(jnp.int32)
        gt_cum = plsc.cumsum(gt_i)
        eq_cum = plsc.cumsum(eq_i)
        eq_rank = eq_run + eq_cum - 1
        eq_sel = eq & (eq_rank < need_eq)
        dst = jnp.where(gt, gt_run + gt_cum - 1,
                        jnp.where(eq_sel, g_tot + eq_rank,
                                  _NCAND + wid))
        gt_run = gt_run + _bcast16(gt_cum, 15)
        eq_run = eq_run + _bcast16(eq_cum, 15)

        row = off + iot
        idxf = (base + row).astype(jnp.float32)
        for c, val in ((0, bx_v[pl.ds(off, 16)]),
                       (1, bx_v[pl.ds(_CHUNK + off, 16)]),
                       (2, bx_v[pl.ds(2 * _CHUNK + off, 16)]),
                       (3, bx_v[pl.ds(3 * _CHUNK + off, 16)]),
                       (4, s16),
                       (5, idxf)):
            plsc.store_scatter(rec_v, [row, zer + c], val)
        dst_v[k // 8, pl.ds((k % 8) * 16, 16)] = dst

    cps = [pltpu.async_copy(rec_v.at[pl.ds(j * 128, 128)],
                            out_hbm.at[dst_v.at[j]], sem)
           for j in range(_CHUNK // 128)]
    for cp in cps:
        cp.wait()


def _nms_core(recs_ref, out_ref, pa_scr, pb_scr, so_scr, soT_scr):
    nblk = _NCAND // _BLK
    si_rows = jnp.transpose(recs_ref[...][:, 4:6])

    def rnk(b, rrow):
        sl = pl.ds(b * _BLK, _BLK)
        s_col = recs_ref[sl, 4:5]
        i_col = recs_ref[sl, 5:6]
        s_row = si_rows[0:1, :]
        i_row = si_rows[1:2, :]
        c = jnp.where((s_col > s_row) |
                      ((s_col == s_row) & (i_col < i_row)), 1.0, 0.0)
        return rrow + jnp.sum(c, axis=0, keepdims=True)

    rank_row = lax.fori_loop(0, nblk, rnk,
                             jnp.zeros((1, _NCAND), jnp.float32))
    rank_i = rank_row.astype(jnp.int32)

    def mkperm(b, carry):
        sl = pl.ds(b * _BLK, _BLK)
        isub = lax.broadcasted_iota(jnp.int32, (_BLK, _NCAND), 0) + b * _BLK
        pa_scr[sl, :] = jnp.where(
            isub == jnp.broadcast_to(rank_i, (_BLK, _NCAND)),
            1.0, 0.0).astype(jnp.bfloat16)
        return carry

    lax.fori_loop(0, nblk, mkperm, 0)
    recs = recs_ref[...]
    r_hi = recs.astype(jnp.bfloat16)
    rem = recs - r_hi.astype(jnp.float32)
    r_mid = rem.astype(jnp.bfloat16)
    r_lo = (rem - r_mid.astype(jnp.float32)).astype(jnp.bfloat16)
    pa = pa_scr[...]
    dn = (((1,), (0,)), ((), ()))
    sorted_v = (
        lax.dot_general(pa, r_hi, dn, preferred_element_type=jnp.float32) +
        lax.dot_general(pa, r_mid, dn, preferred_element_type=jnp.float32) +
        lax.dot_general(pa, r_lo, dn, preferred_element_type=jnp.float32))
    so_scr[...] = sorted_v
    soT_scr[0:4, :] = jnp.transpose(sorted_v[:, 0:4])

    for rb in range(nblk):
        c0 = rb * _BLK
        cw = _NCAND - c0
        rsl = pl.ds(c0, _BLK)
        csl = pl.ds(c0, cw)
        bx = so_scr[rsl, :]
        x1c = bx[:, 0:1]
        y1c = bx[:, 1:2]
        x2c = bx[:, 2:3]
        y2c = bx[:, 3:4]
        area_c = (x2c - x1c) * (y2c - y1c)
        x1r = soT_scr[0:1, csl]
        y1r = soT_scr[1:2, csl]
        x2r = soT_scr[2:3, csl]
        y2r = soT_scr[3:4, csl]
        area_r = (x2r - x1r) * (y2r - y1r)
        w = jnp.maximum(jnp.minimum(x2c, x2r) - jnp.maximum(x1c, x1r), 0.0)
        h = jnp.maximum(jnp.minimum(y2c, y2r) - jnp.maximum(y1c, y1r), 0.0)
        inter = w * h
        union = area_c + area_r - inter
        iou = inter / jnp.maximum(union, 1e-9)
        jrow = lax.broadcasted_iota(jnp.int32, (_BLK, cw), 0) + c0
        icol = lax.broadcasted_iota(jnp.int32, (_BLK, cw), 1) + c0
        pa_scr[rsl, csl] = jnp.where(
            (iou > _THR) & (jrow < icol), 1.0, 0.0).astype(jnp.bfloat16)
        pb_scr[rsl, csl] = jnp.where(
            jrow <= icol, 1.0, 0.0).astype(jnp.bfloat16)
        if c0:
            zer = jnp.zeros((_BLK, c0), jnp.bfloat16)
            pa_scr[rsl, pl.ds(0, c0)] = zer
            pb_scr[rsl, pl.ds(0, c0)] = zer

    S = pa_scr[...]

    def cond(c):
        return c[1]

    def body(c):
        keep, _ = c
        cnt = lax.dot_general(keep.astype(jnp.bfloat16), S,
                              (((1,), (0,)), ((), ())),
                              preferred_element_type=jnp.float32)
        keep_new = jnp.where(cnt == 0.0, 1.0, 0.0)
        return keep_new, jnp.sum(jnp.abs(keep_new - keep)) > 0.0

    keep, _ = lax.while_loop(cond, body,
                             (jnp.ones((8, _NCAND), jnp.float32),
                              jnp.array(True)))

    Lm = pb_scr[...]
    kri = lax.dot_general(keep.astype(jnp.bfloat16), Lm,
                          (((1,), (0,)), ((), ())),
                          preferred_element_type=jnp.float32)
    sri = lax.dot_general((1.0 - keep).astype(jnp.bfloat16), Lm,
                          (((1,), (0,)), ((), ())),
                          preferred_element_type=jnp.float32)
    nk = jnp.sum(keep[0:1, :])
    rank_kept = kri - keep
    rank_supp = nk + (sri - (1.0 - keep))
    out_rank = jnp.where(keep > 0.5, rank_kept, rank_supp)
    oh_rank = out_rank[0:1, :].astype(jnp.int32)
    iota_r = lax.broadcasted_iota(jnp.int32, (128, _NCAND), 0)
    oh = jnp.where(
        iota_r == jnp.broadcast_to(oh_rank, (128, _NCAND)), 1.0, 0.0)
    out = lax.dot_general(oh, so_scr[...], (((1,), (0,)), ((), ())),
                          preferred_element_type=jnp.float32,
                          precision=lax.Precision.HIGHEST)
    nk_i = nk.astype(jnp.int32)
    rr = lax.broadcasted_iota(jnp.int32, (128, 16), 0)
    cc = lax.broadcasted_iota(jnp.int32, (128, 16), 1)
    out_ref[...] = jnp.where((rr >= nk_i) & (cc == 4), -jnp.inf, out)


def kernel(boxes, scores):
    s_pad = jnp.concatenate(
        [scores, jnp.full((_NPAD - _N,), -1.0, jnp.float32)])
    bits = lax.bitcast_convert_type(s_pad, jnp.int32)
    b_pad = jnp.concatenate(
        [boxes, jnp.zeros((_NPAD - _N, 4), jnp.float32)], axis=0)

    meta, counts = pl.pallas_call(
        _thresh_core,
        out_shape=(jax.ShapeDtypeStruct((1, 128), jnp.float32),
                   jax.ShapeDtypeStruct((_NTILE, 8), jnp.float32)),
    )(bits.reshape(8, _NPAD // 8), bits.reshape(_NTILE, _CHUNK))

    mesh = plsc.VectorSubcoreMesh(core_axis_name="c", subcore_axis_name="s")
    recs = functools.partial(
        pl.kernel, mesh=mesh,
        compiler_params=pltpu.CompilerParams(needs_layout_passes=False,
                                             use_tc_tiling_on_sc=False),
        out_type=jax.ShapeDtypeStruct((_NCAND + _NTILE, 16), jnp.float32),
        scratch_types=[
            pltpu.VMEM((_CHUNK,), jnp.float32),
            pltpu.VMEM((4 * _CHUNK,), jnp.float32),
            pltpu.VMEM((128,), jnp.float32),
            pltpu.VMEM((_NTILE, 8), jnp.float32),
            pltpu.VMEM((_CHUNK, 16), jnp.float32),
            pltpu.VMEM((_CHUNK // 128, 128), jnp.int32),
            pltpu.SemaphoreType.DMA,
        ],
    )(_sc_compact)(s_pad, b_pad.T.reshape(-1), meta.reshape(-1), counts)

    recs = recs[:_NCAND]
    out = pl.pallas_call(
        _nms_core,
        out_shape=jax.ShapeDtypeStruct((128, 16), jnp.float32),
        scratch_shapes=[
            pltpu.VMEM((_NCAND, _NCAND), jnp.bfloat16),
            pltpu.VMEM((_NCAND, _NCAND), jnp.bfloat16),
            pltpu.VMEM((_NCAND, 16), jnp.float32),
            pltpu.VMEM((8, _NCAND), jnp.float32),
        ],
    )(recs)
    return out[:_NOUT, :4], out[:_NOUT, 4]
